# baseline (speedup 1.0000x reference)
import jax
import jax.numpy as jnp
from jax.experimental import pallas as pl
from jax.experimental.pallas import tpu as pltpu

N = 100000
S = 512
D = 512
HPAD = 128
B = 2000
NSTEP = N // B // 2
W = 32


def _stream_update(x_ref, b_ref, lo_ref, hi_ref, j, gw1_ref, p_ref,
                   m_ref, d_ref, acc_ref):
    xb = x_ref[...]
    h = jnp.maximum(
        jnp.dot(xb, gw1_ref[...], preferred_element_type=jnp.float32)
        + p_ref[0:1, :], 0.0)
    g_row = jax.lax.dot_general(
        p_ref[1:2, :], h, (((1,), (1,)), ((), ())),
        preferred_element_type=jnp.float32)
    bb = b_ref[0, 0, :].reshape(1, B)

    s0a = jnp.minimum((lo_ref[j] // 8) * 8, S - W)
    span_ok = (hi_ref[j] - s0a) < W

    @pl.when(span_ok)
    def _fast():
        bb_rel = bb - s0a
        seg = jax.lax.broadcasted_iota(jnp.int32, (W, B), 0)
        O2 = seg == bb_rel
        gmax = jnp.max(g_row)
        e_raw = jnp.exp(g_row - gmax)
        Oe = jnp.where(O2, e_raw, 0.0)
        dsum = jnp.sum(Oe, axis=1, keepdims=True)
        m_old = m_ref[pl.ds(s0a, W), :]
        m_new = jnp.maximum(m_old, gmax)
        scale = jnp.where(jnp.isfinite(m_old), jnp.exp(m_old - m_new), 0.0)
        corr = jnp.exp(gmax - m_new)
        d_ref[pl.ds(s0a, W), :] = (
            d_ref[pl.ds(s0a, W), :] * scale + corr * dsum)
        acc_ref[pl.ds(s0a, W), :] = (
            acc_ref[pl.ds(s0a, W), :] * scale
            + corr * jnp.dot(Oe, xb, preferred_element_type=jnp.float32))
        m_ref[pl.ds(s0a, W), :] = m_new

    @pl.when(jnp.logical_not(span_ok))
    def _slow():
        seg = jax.lax.broadcasted_iota(jnp.int32, (S, B), 0)
        O2 = seg == bb
        mb = jnp.max(jnp.where(O2, g_row, -jnp.inf), axis=1, keepdims=True)
        m_old = m_ref[...]
        m_new = jnp.maximum(m_old, mb)
        scale = jnp.where(jnp.isfinite(m_old), jnp.exp(m_old - m_new), 0.0)
        gath = jnp.sum(jnp.where(O2, m_new, 0.0), axis=0, keepdims=True)
        e_row = jnp.exp(g_row - gath)
        Oe = jnp.where(O2, e_row, 0.0)
        d_ref[...] = d_ref[...] * scale + jnp.sum(Oe, axis=1, keepdims=True)
        acc_ref[...] = acc_ref[...] * scale + jnp.dot(
            Oe, xb, preferred_element_type=jnp.float32)
        m_ref[...] = m_new


def _fused_kernel(lo_ref, hi_ref, x1_ref, x2_ref, b1_ref, b2_ref, gw1_ref,
                  mw1_ref, p_ref, out_ref, m_ref, d_ref, acc_ref):
    i = pl.program_id(0)

    @pl.when(i == 0)
    def _init():
        m_ref[...] = jnp.full((S, 1), -jnp.inf, jnp.float32)
        d_ref[...] = jnp.zeros((S, 1), jnp.float32)
        acc_ref[...] = jnp.zeros((S, D), jnp.float32)

    _stream_update(x1_ref, b1_ref, lo_ref, hi_ref, i, gw1_ref, p_ref,
                   m_ref, d_ref, acc_ref)
    _stream_update(x2_ref, b2_ref, lo_ref, hi_ref, NSTEP + i, gw1_ref, p_ref,
                   m_ref, d_ref, acc_ref)

    @pl.when(i == NSTEP - 1)
    def _finish():
        hg = acc_ref[...] / (d_ref[...] + 1e-16)
        h2 = jnp.maximum(
            jnp.dot(hg, mw1_ref[...], preferred_element_type=jnp.float32)
            + p_ref[2:3, :], 0.0)
        logit = jnp.sum(h2 * p_ref[3:4, :], axis=1, keepdims=True)
        out_ref[...] = logit + p_ref[4:5, 0:1]


@jax.jit
def _run(x, batch3, lo, hi, gw1p, mw1p, params):
    return pl.pallas_call(
        _fused_kernel,
        grid=(NSTEP,),
        in_specs=[
            pl.BlockSpec(memory_space=pltpu.SMEM),
            pl.BlockSpec(memory_space=pltpu.SMEM),
            pl.BlockSpec((B, D), lambda i: (i, 0)),
            pl.BlockSpec((B, D), lambda i: (NSTEP + i, 0)),
            pl.BlockSpec((1, 1, B), lambda i: (i, 0, 0)),
            pl.BlockSpec((1, 1, B), lambda i: (NSTEP + i, 0, 0)),
            pl.BlockSpec((D, HPAD), lambda i: (0, 0)),
            pl.BlockSpec((D, HPAD), lambda i: (0, 0)),
            pl.BlockSpec((8, HPAD), lambda i: (0, 0)),
        ],
        out_specs=pl.BlockSpec((S, 1), lambda i: (0, 0)),
        out_shape=jax.ShapeDtypeStruct((S, 1), jnp.float32),
        scratch_shapes=[
            pltpu.VMEM((S, 1), jnp.float32),
            pltpu.VMEM((S, 1), jnp.float32),
            pltpu.VMEM((S, D), jnp.float32),
        ],
    )(lo, hi, x, x, batch3, batch3, gw1p, mw1p, params)


def kernel(x, batch, gate_w1, gate_b1, gate_w2, gate_b2,
           mlp_w1, mlp_b1, mlp_w2, mlp_b2):
    hid = gate_w1.shape[1]
    batch32 = batch.astype(jnp.int32)
    batch3 = batch32.reshape(N // B, 1, B)
    lo = batch32[0::B]
    hi = batch32[B - 1::B]
    gw1p = jnp.zeros((D, HPAD), jnp.float32).at[:, :hid].set(gate_w1)
    mw1p = jnp.zeros((D, HPAD), jnp.float32).at[:, :hid].set(mlp_w1)
    params = (
        jnp.zeros((8, HPAD), jnp.float32)
        .at[0, :hid].set(gate_b1)
        .at[1, :hid].set(gate_w2[:, 0])
        .at[2, :hid].set(mlp_b1)
        .at[3, :hid].set(mlp_w2[:, 0])
        .at[4, 0].set(mlp_b2[0])
    )
    return _run(x, batch3, lo, hi, gw1p, mw1p, params)


# speedup vs baseline: 1.0812x; 1.0812x over previous
import jax
import jax.numpy as jnp
from jax.experimental import pallas as pl
from jax.experimental.pallas import tpu as pltpu

N = 100000
S = 512
D = 512
DH = D // 2
HPAD = 128
B = 4000
NB = N // B
W = 64


def _fused_kernel(lo_ref, hi_ref, x1_ref, x2_ref, b_ref, gw1a_ref, gw1b_ref,
                  mw1_ref, p_ref, out_ref, m_ref, d_ref, acc_ref):
    i = pl.program_id(0)

    @pl.when(i == 0)
    def _init():
        m_ref[...] = jnp.full((S, 1), -jnp.inf, jnp.float32)
        d_ref[...] = jnp.zeros((S, 1), jnp.float32)
        acc_ref[...] = jnp.zeros((S, D), jnp.float32)

    x1 = x1_ref[...]
    x2 = x2_ref[...]
    h = jnp.maximum(
        jnp.dot(x1, gw1a_ref[...], preferred_element_type=jnp.float32)
        + jnp.dot(x2, gw1b_ref[...], preferred_element_type=jnp.float32)
        + p_ref[0:1, :], 0.0)
    g_row = jax.lax.dot_general(
        p_ref[1:2, :], h, (((1,), (1,)), ((), ())),
        preferred_element_type=jnp.float32)
    bb = b_ref[0, 0, :].reshape(1, B)

    s0 = lo_ref[i]
    s0a = jnp.minimum((s0 // 8) * 8, S - W)
    span_ok = (hi_ref[i] - s0a) < W

    @pl.when(span_ok)
    def _fast():
        bb_rel = bb - s0a
        seg = jax.lax.broadcasted_iota(jnp.int32, (W, B), 0)
        O2 = seg == bb_rel
        gmax = jnp.max(g_row)
        e_raw = jnp.exp(g_row - gmax)
        Oe = jnp.where(O2, e_raw, 0.0)
        dsum = jnp.sum(Oe, axis=1, keepdims=True)
        m_old = m_ref[pl.ds(s0a, W), :]
        m_new = jnp.maximum(m_old, gmax)
        scale = jnp.where(jnp.isfinite(m_old), jnp.exp(m_old - m_new), 0.0)
        corr = jnp.exp(gmax - m_new)
        d_ref[pl.ds(s0a, W), :] = (
            d_ref[pl.ds(s0a, W), :] * scale + corr * dsum)
        acc_ref[pl.ds(s0a, W), 0:DH] = (
            acc_ref[pl.ds(s0a, W), 0:DH] * scale
            + corr * jnp.dot(Oe, x1, preferred_element_type=jnp.float32))
        acc_ref[pl.ds(s0a, W), DH:D] = (
            acc_ref[pl.ds(s0a, W), DH:D] * scale
            + corr * jnp.dot(Oe, x2, preferred_element_type=jnp.float32))
        m_ref[pl.ds(s0a, W), :] = m_new

    @pl.when(jnp.logical_not(span_ok))
    def _slow():
        seg = jax.lax.broadcasted_iota(jnp.int32, (S, B), 0)
        O2 = seg == bb
        mb = jnp.max(jnp.where(O2, g_row, -jnp.inf), axis=1, keepdims=True)
        m_old = m_ref[...]
        m_new = jnp.maximum(m_old, mb)
        scale = jnp.where(jnp.isfinite(m_old), jnp.exp(m_old - m_new), 0.0)
        gath = jnp.sum(jnp.where(O2, m_new, 0.0), axis=0, keepdims=True)
        e_row = jnp.exp(g_row - gath)
        Oe = jnp.where(O2, e_row, 0.0)
        d_ref[...] = d_ref[...] * scale + jnp.sum(Oe, axis=1, keepdims=True)
        acc_ref[:, 0:DH] = acc_ref[:, 0:DH] * scale + jnp.dot(
            Oe, x1, preferred_element_type=jnp.float32)
        acc_ref[:, DH:D] = acc_ref[:, DH:D] * scale + jnp.dot(
            Oe, x2, preferred_element_type=jnp.float32)
        m_ref[...] = m_new

    @pl.when(i == NB - 1)
    def _finish():
        hg = acc_ref[...] / (d_ref[...] + 1e-16)
        h2 = jnp.maximum(
            jnp.dot(hg, mw1_ref[...], preferred_element_type=jnp.float32)
            + p_ref[2:3, :], 0.0)
        logit = jnp.sum(h2 * p_ref[3:4, :], axis=1, keepdims=True)
        out_ref[...] = logit + p_ref[4:5, 0:1]


@jax.jit
def _run(x, batch3, lo, hi, gw1a, gw1b, mw1p, params):
    return pl.pallas_call(
        _fused_kernel,
        grid=(NB,),
        in_specs=[
            pl.BlockSpec(memory_space=pltpu.SMEM),
            pl.BlockSpec(memory_space=pltpu.SMEM),
            pl.BlockSpec((B, DH), lambda i: (i, 0)),
            pl.BlockSpec((B, DH), lambda i: (i, 1)),
            pl.BlockSpec((1, 1, B), lambda i: (i, 0, 0)),
            pl.BlockSpec((DH, HPAD), lambda i: (0, 0)),
            pl.BlockSpec((DH, HPAD), lambda i: (0, 0)),
            pl.BlockSpec((D, HPAD), lambda i: (0, 0)),
            pl.BlockSpec((8, HPAD), lambda i: (0, 0)),
        ],
        out_specs=pl.BlockSpec((S, 1), lambda i: (0, 0)),
        out_shape=jax.ShapeDtypeStruct((S, 1), jnp.float32),
        scratch_shapes=[
            pltpu.VMEM((S, 1), jnp.float32),
            pltpu.VMEM((S, 1), jnp.float32),
            pltpu.VMEM((S, D), jnp.float32),
        ],
    )(lo, hi, x, x, batch3, gw1a, gw1b, mw1p, params)


def kernel(x, batch, gate_w1, gate_b1, gate_w2, gate_b2,
           mlp_w1, mlp_b1, mlp_w2, mlp_b2):
    hid = gate_w1.shape[1]
    batch32 = batch.astype(jnp.int32)
    batch3 = batch32.reshape(NB, 1, B)
    lo = batch32[0::B]
    hi = batch32[B - 1::B]
    gw1p = jnp.zeros((D, HPAD), jnp.float32).at[:, :hid].set(gate_w1)
    mw1p = jnp.zeros((D, HPAD), jnp.float32).at[:, :hid].set(mlp_w1)
    params = (
        jnp.zeros((8, HPAD), jnp.float32)
        .at[0, :hid].set(gate_b1)
        .at[1, :hid].set(gate_w2[:, 0])
        .at[2, :hid].set(mlp_b1)
        .at[3, :hid].set(mlp_w2[:, 0])
        .at[4, 0].set(mlp_b2[0])
    )
    return _run(x, batch3, lo, hi, gw1p[:DH], gw1p[DH:], mw1p, params)
